# Initial kernel scaffold; baseline (speedup 1.0000x reference)
import jax
import jax.numpy as jnp
from jax.experimental import pallas as pl


def kernel(x, edge_attr, params, edge_index, batch):
    raise NotImplementedError("write your pallas kernel here")


# speedup vs baseline: 2.6494x; 2.6494x over previous
import functools

import jax
import jax.numpy as jnp
from jax.experimental import pallas as pl
from jax.experimental.pallas import tpu as pltpu
from jax.experimental.pallas import tpu_sc as plsc

_H = 128
_G = 64
_STEPS = 6
_EW = 128
_NW = 32
_F32 = jnp.float32


def _proj_body(x_ref, wp_ref, bp_ref, ws_ref, wd_ref, bm_ref,
               h_ref, a_ref, b_ref):
    x = x_ref[...]
    h = jnp.dot(x, wp_ref[...], preferred_element_type=_F32) + bp_ref[...]
    h = jnp.where(h > 0, h, jnp.exp(jnp.minimum(h, 0.0)) - 1.0)
    h_ref[...] = h
    a_ref[...] = jnp.dot(h, ws_ref[...], preferred_element_type=_F32)
    b_ref[...] = jnp.dot(h, wd_ref[...], preferred_element_type=_F32) + bm_ref[...]


def _proj(x, wp, bp, ws, wd, bm):
    n = x.shape[0]
    blk = 1000
    full = pl.BlockSpec((_H, _H), lambda i: (0, 0))
    bias = pl.BlockSpec((1, _H), lambda i: (0, 0))
    row = pl.BlockSpec((blk, _H), lambda i: (i, 0))
    return pl.pallas_call(
        _proj_body,
        grid=(n // blk,),
        in_specs=[row, full, bias, full, full, bias],
        out_specs=[row, row, row],
        out_shape=[jax.ShapeDtypeStruct((n, _H), _F32)] * 3,
    )(x, wp, bp, ws, wd, bm)


def _upd_body(h_ref, g0_ref, g1_ref, wu_ref, bu_ref, ws_ref, wd_ref, bm_ref,
              hn_ref, a_ref, b_ref):
    agg = g0_ref[0] + g1_ref[0]
    hn = (h_ref[...] + jnp.dot(agg, wu_ref[...], preferred_element_type=_F32)
          + bu_ref[...])
    hn_ref[...] = hn
    a_ref[...] = jnp.dot(hn, ws_ref[...], preferred_element_type=_F32)
    b_ref[...] = jnp.dot(hn, wd_ref[...], preferred_element_type=_F32) + bm_ref[...]


def _upd(h, g, wu, bu, ws, wd, bm):
    n = h.shape[0]
    blk = 1000
    full = pl.BlockSpec((_H, _H), lambda i: (0, 0))
    bias = pl.BlockSpec((1, _H), lambda i: (0, 0))
    row = pl.BlockSpec((blk, _H), lambda i: (i, 0))
    g0s = pl.BlockSpec((1, blk, _H), lambda i: (0, i, 0))
    g1s = pl.BlockSpec((1, blk, _H), lambda i: (1, i, 0))
    return pl.pallas_call(
        _upd_body,
        grid=(n // blk,),
        in_specs=[row, g0s, g1s, full, bias, full, full, bias],
        out_specs=[row, row, row],
        out_shape=[jax.ShapeDtypeStruct((n, _H), _F32)] * 3,
    )(h, g, g, wu, bu, ws, wd, bm)


def _upd_last_body(h_ref, g0_ref, g1_ref, wu_ref, bu_ref, hn_ref):
    agg = g0_ref[0] + g1_ref[0]
    hn_ref[...] = (h_ref[...]
                   + jnp.dot(agg, wu_ref[...], preferred_element_type=_F32)
                   + bu_ref[...])


def _upd_last(h, g, wu, bu):
    n = h.shape[0]
    blk = 1000
    full = pl.BlockSpec((_H, _H), lambda i: (0, 0))
    bias = pl.BlockSpec((1, _H), lambda i: (0, 0))
    row = pl.BlockSpec((blk, _H), lambda i: (i, 0))
    g0s = pl.BlockSpec((1, blk, _H), lambda i: (0, i, 0))
    g1s = pl.BlockSpec((1, blk, _H), lambda i: (1, i, 0))
    return pl.pallas_call(
        _upd_last_body,
        grid=(n // blk,),
        in_specs=[row, g0s, g1s, full, bias],
        out_specs=row,
        out_shape=jax.ShapeDtypeStruct((n, _H), _F32),
    )(h, g, g, wu, bu)


def _edgec_body(ea_ref, w_ref, c0_ref, c1_ref, c2_ref):
    c = jnp.dot(ea_ref[...], w_ref[...], preferred_element_type=_F32)
    c0_ref[...] = c[:, :_H]
    c1_ref[...] = c[:, _H:2 * _H]
    c2_ref[...] = c[:, 2 * _H:]


def _edgec(ea, wcat):
    ep, ef = ea.shape
    blk = 2048
    row_in = pl.BlockSpec((blk, ef), lambda i: (i, 0))
    wfull = pl.BlockSpec((ef, 3 * _H), lambda i: (0, 0))
    row_out = pl.BlockSpec((blk, _H), lambda i: (i, 0))
    return pl.pallas_call(
        _edgec_body,
        grid=(ep // blk,),
        in_specs=[row_in, wfull],
        out_specs=[row_out] * 3,
        out_shape=[jax.ShapeDtypeStruct((ep, _H), _F32)] * 3,
    )(ea, wcat)


def _s2s_body(h_ref, batch_ref, wih_ref, whh_ref, lb_ref,
              w1_ref, b1_ref, gam_ref, bet_ref, w2_ref, b2_ref, out_ref):
    h = h_ref[...]
    n = h.shape[0]
    seg = jax.lax.broadcasted_iota(jnp.int32, (n, _G), 1)
    s_hot = jnp.where(batch_ref[...] == seg, 1.0, 0.0).astype(_F32)
    s_big = (s_hot - 1.0) * 1e30
    q_star = jnp.zeros((_G, 2 * _H), _F32)
    ht = jnp.zeros((_G, _H), _F32)
    ct = jnp.zeros((_G, _H), _F32)
    for _ in range(_STEPS):
        gates = (jnp.dot(q_star, wih_ref[...], preferred_element_type=_F32)
                 + jnp.dot(ht, whh_ref[...], preferred_element_type=_F32)
                 + lb_ref[...])
        gi = jax.nn.sigmoid(gates[:, :_H])
        gf = jax.nn.sigmoid(gates[:, _H:2 * _H])
        gg = jnp.tanh(gates[:, 2 * _H:3 * _H])
        go = jax.nn.sigmoid(gates[:, 3 * _H:])
        ct = gf * ct + gi * gg
        ht = go * jnp.tanh(ct)
        qb = jnp.dot(s_hot, ht, preferred_element_type=_F32,
                     precision=jax.lax.Precision.HIGHEST)
        e = jnp.sum(h * qb, axis=1, keepdims=True)
        emask = s_hot * e + s_big
        emax = jnp.max(emask, axis=0, keepdims=True)
        eegrid = jnp.exp(emask - emax) * s_hot
        denom = jnp.maximum(jnp.sum(eegrid, axis=0, keepdims=True), 1e-30)
        agrid = eegrid / denom
        r = jax.lax.dot_general(agrid, h, (((0,), (0,)), ((), ())),
                                preferred_element_type=_F32,
                                precision=jax.lax.Precision.HIGHEST)
        q_star = jnp.concatenate([ht, r], axis=1)
    y = jnp.dot(q_star, w1_ref[...], preferred_element_type=_F32) + b1_ref[...]
    mu = jnp.mean(y, axis=1, keepdims=True)
    var = jnp.mean((y - mu) ** 2, axis=1, keepdims=True)
    y = (y - mu) * jax.lax.rsqrt(var + 1e-5) * gam_ref[...] + bet_ref[...]
    y = jnp.maximum(y, 0.0)
    out_ref[...] = jnp.dot(y, w2_ref[...], preferred_element_type=_F32) + b2_ref[...]


def _s2s(h, batch2d, lstm, mlp):
    n = h.shape[0]

    def spec(shape):
        return pl.BlockSpec(shape, lambda: tuple(0 for _ in shape))

    return pl.pallas_call(
        _s2s_body,
        in_specs=[spec((n, _H)), spec((n, 1)), spec((2 * _H, 4 * _H)),
                  spec((_H, 4 * _H)), spec((1, 4 * _H)),
                  spec((2 * _H, _H)), spec((1, _H)), spec((1, _H)),
                  spec((1, _H)), spec((_H, 1)), spec((1, 1))],
        out_specs=spec((_G, 1)),
        out_shape=jax.ShapeDtypeStruct((_G, 1), _F32),
    )(h, batch2d, lstm['W_ih'], lstm['W_hh'], lstm['b'].reshape(1, -1),
      mlp['W1'], mlp['b1'].reshape(1, -1), mlp['gamma'].reshape(1, -1),
      mlp['beta'].reshape(1, -1), mlp['W2'], mlp['b2'].reshape(1, -1))


def _edge_pass(a_tab, b_tab, c_rows, src, dst):
    n = a_tab.shape[0]
    ep = src.shape[0]
    nchunks = ep // (_NW * _EW)
    epw = nchunks * _EW
    span = ((n + 16 + 15) // 16 + 7) // 8 * 8
    np2 = 16 * span
    mesh = plsc.VectorSubcoreMesh(core_axis_name="c", subcore_axis_name="s",
                                  num_cores=2, num_subcores=16)
    nvec = _H // 16

    def body(a_hbm, b_hbm, c_hbm, src_hbm, dst_hbm, o_hbm,
             idx_s, idx_d, rows_a, rows_b, rows_m, agg_sh):
        ci = jax.lax.axis_index("c")
        si = jax.lax.axis_index("s")

        @pl.loop(0, _EW)
        def _(r):
            for j in range(nvec):
                rows_m.at[pl.ds(r, 1), pl.ds(j * 16, 16)][...] = (
                    jnp.zeros((1, 16), _F32))
        zb = si * span
        zfull, zrem = divmod(span, _EW)
        for k in range(zfull):
            pltpu.sync_copy(rows_m, agg_sh.at[pl.ds(zb + k * _EW, _EW)])
        if zrem:
            pltpu.sync_copy(rows_m.at[pl.ds(0, zrem)],
                            agg_sh.at[pl.ds(zb + zfull * _EW, zrem)])
        plsc.subcore_barrier()

        wid = si * 2 + ci
        wbase = wid * epw

        @pl.loop(0, nchunks)
        def _(i):
            base = wbase + i * _EW
            pltpu.sync_copy(src_hbm.at[pl.ds(base, _EW)], idx_s)
            pltpu.sync_copy(dst_hbm.at[pl.ds(base, _EW)], idx_d)
            pltpu.sync_copy(a_hbm.at[idx_s], rows_a)
            pltpu.sync_copy(b_hbm.at[idx_d], rows_b)
            pltpu.sync_copy(c_hbm.at[pl.ds(base, _EW)], rows_m)

            @pl.loop(0, _EW)
            def _(r):
                for j in range(nvec):
                    sl = (pl.ds(r, 1), pl.ds(j * 16, 16))
                    v = (rows_a.at[sl[0], sl[1]][...]
                         + rows_b.at[sl[0], sl[1]][...]
                         + rows_m.at[sl[0], sl[1]][...])
                    rows_m.at[sl[0], sl[1]][...] = jnp.maximum(v, 0.0)

            pltpu.sync_copy(rows_m, agg_sh.at[idx_d], add=True)

        plsc.subcore_barrier()
        ob = si * span
        pltpu.sync_copy(agg_sh.at[pl.ds(ob, span)],
                        o_hbm.at[ci, pl.ds(ob, span)])

    call = pl.kernel(
        body, mesh=mesh,
        out_type=jax.ShapeDtypeStruct((2, np2, _H), _F32),
        scratch_types=[
            pltpu.VMEM((_EW,), jnp.int32),
            pltpu.VMEM((_EW,), jnp.int32),
            pltpu.VMEM((_EW, _H), _F32),
            pltpu.VMEM((_EW, _H), _F32),
            pltpu.VMEM((_EW, _H), _F32),
            pltpu.VMEM_SHARED((np2, _H), _F32),
        ],
    )
    return call(a_tab, b_tab, c_rows, src, dst)


def kernel(x, edge_attr, params, edge_index, batch):
    n = x.shape[0]
    e = edge_index.shape[1]
    ep = ((e + _NW * _EW - 1) // (_NW * _EW)) * (_NW * _EW)
    pad = ep - e

    src = edge_index[0]
    dst = edge_index[1]
    src_p = jnp.concatenate([src, jnp.zeros((pad,), src.dtype)])
    dst_p = jnp.concatenate([dst, jnp.full((pad,), n, dst.dtype)])
    ea_p = jnp.concatenate(
        [edge_attr, jnp.zeros((pad, edge_attr.shape[1]), edge_attr.dtype)], axis=0)

    layers = params['layers']
    ws = [p['W_msg'][:_H] for p in layers]
    wd = [p['W_msg'][_H:2 * _H] for p in layers]
    wcat = jnp.concatenate([p['W_msg'][2 * _H:] for p in layers], axis=1)
    bm = [p['b_msg'].reshape(1, -1) for p in layers]

    h, a_tab, b_tab = _proj(x, params['W_proj'], params['b_proj'].reshape(1, -1),
                            ws[0], wd[0], bm[0])
    c_all = _edgec(ea_p, wcat)

    for l in range(len(layers)):
        g = _edge_pass(a_tab, b_tab, c_all[l], src_p, dst_p)
        wu = layers[l]['W_upd']
        bu = layers[l]['b_upd'].reshape(1, -1)
        if l + 1 < len(layers):
            h, a_tab, b_tab = _upd(h, g, wu, bu,
                                   ws[l + 1], wd[l + 1], bm[l + 1])
        else:
            h = _upd_last(h, g, wu, bu)

    batch2d = batch.reshape(n, 1).astype(jnp.int32)
    return _s2s(h, batch2d, params['lstm'], params['mlp'])
